# baseline (speedup 1.0000x reference)
import functools

import jax
import jax.numpy as jnp
from jax import lax
from jax.experimental import pallas as pl
from jax.experimental.pallas import tpu as pltpu
from jax.experimental.pallas import tpu_sc as plsc

HIDDEN = 4096
N_TOK = 16384
SCS_TOK = 2048
TEC_TOK = N_TOK - SCS_TOK
NUM_CORES = 2
NUM_SUBCORES = 16
NW = NUM_CORES * NUM_SUBCORES
B_PER_W = TEC_TOK // NW
CHUNK = 8
N_CHUNKS = B_PER_W // CHUNK
NBUF = 2
NSEM = 8
SCS_PER_CORE = SCS_TOK // NUM_CORES


def _tec_fn(tec_ids_hbm, scs_ids_hbm, table_hbm, out_hbm):
    wid = lax.axis_index("s") * NUM_CORES + lax.axis_index("c")
    base = wid * B_PER_W

    def scoped(idx_v, rows_v, *sems):
        gsem = list(sems[:NBUF])
        osem = list(sems[NBUF:])
        pltpu.sync_copy(tec_ids_hbm.at[pl.ds(wid * N_CHUNKS, N_CHUNKS)], idx_v)

        def g_desc(j, b):
            return pltpu.make_async_copy(
                table_hbm.at[idx_v.at[j]], rows_v.at[b], gsem[b]
            )

        def o_desc(j, b):
            return pltpu.make_async_copy(
                rows_v.at[b],
                out_hbm.at[pl.ds(base + j * CHUNK, CHUNK)],
                osem[b],
            )

        for b in range(NBUF):
            g_desc(b, b).start()

        def outer(i, carry):
            j0 = i * NBUF
            for b in range(NBUF):
                j = j0 + b
                g_desc(j, b).wait()
                o_desc(j, b).start()

                @pl.when(j + NBUF < N_CHUNKS)
                def _():
                    o_desc(j, b).wait()
                    g_desc(j + NBUF, b).start()

            return carry

        lax.fori_loop(0, N_CHUNKS // NBUF, outer, 0)

        for b in range(NBUF):
            o_desc(N_CHUNKS - NBUF + b, b).wait()

    pl.run_scoped(
        scoped,
        pltpu.VMEM((N_CHUNKS, CHUNK), jnp.int32),
        pltpu.VMEM((NBUF, CHUNK, HIDDEN), jnp.float32),
        *([pltpu.SemaphoreType.DMA] * (2 * NBUF)),
    )


def _scs_fn(tec_ids_hbm, scs_ids_hbm, table_hbm, out_hbm):
    cid = lax.axis_index("c")
    base = cid * SCS_PER_CORE
    out_base = TEC_TOK + base

    def scoped(ids_s, *sems):
        pltpu.sync_copy(scs_ids_hbm.at[pl.ds(base, SCS_PER_CORE)], ids_s)

        def row_copy(i, b):
            idx = ids_s[i]
            pltpu.make_async_copy(
                table_hbm.at[pl.ds(idx, 1)],
                out_hbm.at[pl.ds(out_base + i, 1)],
                sems[b],
            ).start()

        def drain(b):
            pltpu.make_async_copy(
                table_hbm.at[pl.ds(0, 1)],
                out_hbm.at[pl.ds(out_base, 1)],
                sems[b],
            ).wait()

        def outer(o, carry):
            for b in range(NSEM):
                i = o * NSEM + b

                @pl.when(o > 0)
                def _():
                    drain(b)

                row_copy(i, b)
            return carry

        lax.fori_loop(0, SCS_PER_CORE // NSEM, outer, 0)

        for b in range(NSEM):
            drain(b)

    pl.run_scoped(
        scoped,
        pltpu.SMEM((SCS_PER_CORE,), jnp.int32),
        *([pltpu.SemaphoreType.DMA] * NSEM),
    )


def _build():
    vec_mesh = plsc.VectorSubcoreMesh(core_axis_name="c", subcore_axis_name="s")
    scs_mesh = plsc.ScalarSubcoreMesh(axis_name="c")
    return pl.kernel(
        body=[_tec_fn, _scs_fn],
        mesh=[vec_mesh, scs_mesh],
        out_type=jax.ShapeDtypeStruct((N_TOK, HIDDEN), jnp.float32),
    )


_emb = _build()


def kernel(input_ids, embed_tokens_weight):
    b, s = input_ids.shape
    ids_flat = input_ids.reshape(-1).astype(jnp.int32)
    tec_ids = ids_flat[:TEC_TOK].reshape(TEC_TOK // CHUNK, CHUNK)
    scs_ids = ids_flat[TEC_TOK:]
    out = _emb(tec_ids, scs_ids, embed_tokens_weight)
    return out.reshape(b, s, HIDDEN)


# speedup vs baseline: 5.0295x; 5.0295x over previous
import functools

import jax
import jax.numpy as jnp
from jax import lax
from jax.experimental import pallas as pl
from jax.experimental.pallas import tpu as pltpu
from jax.experimental.pallas import tpu_sc as plsc

HIDDEN = 4096
N_TOK = 16384
NUM_CORES = 2
NUM_SUBCORES = 16
NW = NUM_CORES * NUM_SUBCORES
B_PER_W = N_TOK // NW
CHUNK = 8
N_CHUNKS = B_PER_W // CHUNK
NBUF = 3


def _build():
    mesh = plsc.VectorSubcoreMesh(core_axis_name="c", subcore_axis_name="s")

    @functools.partial(
        pl.kernel,
        mesh=mesh,
        out_type=jax.ShapeDtypeStruct((N_TOK, HIDDEN), jnp.float32),
        scratch_types=[
            pltpu.VMEM((N_CHUNKS, CHUNK), jnp.int32),
            pltpu.VMEM((NBUF, CHUNK, HIDDEN), jnp.float32),
        ] + [pltpu.SemaphoreType.DMA] * (2 * NBUF),
    )
    def emb(ids_hbm, table_hbm, out_hbm, idx_v, rows_v, *sems):
        gsem = list(sems[:NBUF])
        osem = list(sems[NBUF:])
        wid = lax.axis_index("s") * NUM_CORES + lax.axis_index("c")
        base = wid * B_PER_W
        pltpu.sync_copy(ids_hbm.at[pl.ds(wid * N_CHUNKS, N_CHUNKS)], idx_v)

        def g_desc(j, b):
            return pltpu.make_async_copy(
                table_hbm.at[idx_v.at[j]],
                rows_v.at[b],
                gsem[b],
            )

        def o_desc(j, b):
            return pltpu.make_async_copy(
                rows_v.at[b],
                out_hbm.at[pl.ds(base + j * CHUNK, CHUNK)],
                osem[b],
            )

        for b in range(NBUF):
            g_desc(b, b).start()

        def outer(i, carry):
            j0 = i * NBUF
            for b in range(NBUF):
                j = j0 + b

                @pl.when(j < N_CHUNKS)
                def _():
                    g_desc(j, b).wait()
                    o_desc(j, b).start()

                    @pl.when(j + NBUF < N_CHUNKS)
                    def _():
                        o_desc(j, b).wait()
                        g_desc(j + NBUF, b).start()

            return carry

        n_outer = (N_CHUNKS + NBUF - 1) // NBUF
        lax.fori_loop(0, n_outer, outer, 0)

        for j in range(N_CHUNKS - NBUF, N_CHUNKS):
            o_desc(j, j % NBUF).wait()

    return emb


_emb = _build()


def kernel(input_ids, embed_tokens_weight):
    b, s = input_ids.shape
    ids_flat = input_ids.reshape(N_TOK // CHUNK, CHUNK).astype(jnp.int32)
    out = _emb(ids_flat, embed_tokens_weight)
    return out.reshape(b, s, HIDDEN)
